# Initial kernel scaffold; baseline (speedup 1.0000x reference)
import jax
import jax.numpy as jnp
from jax.experimental import pallas as pl


def kernel(inData, inIndex, inShape):
    raise NotImplementedError("write your pallas kernel here")


# speedup vs baseline: 4.5693x; 4.5693x over previous
import functools

import jax
import jax.numpy as jnp
from jax import lax
from jax.experimental import pallas as pl
from jax.experimental.pallas import tpu as pltpu
from jax.experimental.pallas import tpu_sc as plsc

_LANES = 16


def _sc_gather(table, idx):
    B = idx.shape[0]
    info = plsc.get_sparse_core_info()
    nc, ns = info.num_cores, info.num_subcores
    nw = nc * ns
    b_per_w = B // nw
    t_pad = table.shape[0]

    mesh = plsc.VectorSubcoreMesh(core_axis_name="c", subcore_axis_name="s")

    @functools.partial(
        pl.kernel,
        mesh=mesh,
        out_type=jax.ShapeDtypeStruct((B,), jnp.float32),
        compiler_params=pltpu.CompilerParams(needs_layout_passes=False),
        scratch_types=[
            pltpu.VMEM((t_pad,), jnp.float32),
            pltpu.VMEM((b_per_w,), jnp.int32),
            pltpu.VMEM((b_per_w,), jnp.float32),
        ],
    )
    def k(table_hbm, idx_hbm, out_hbm, table_v, idx_v, out_v):
        wid = lax.axis_index("s") * nc + lax.axis_index("c")
        base = wid * b_per_w
        pltpu.sync_copy(table_hbm, table_v)
        pltpu.sync_copy(idx_hbm.at[pl.ds(base, b_per_w)], idx_v)
        for i in range(b_per_w // _LANES):
            ids = idx_v[pl.ds(i * _LANES, _LANES)]
            out_v[pl.ds(i * _LANES, _LANES)] = plsc.load_gather(table_v, [ids])
        pltpu.sync_copy(out_v, out_hbm.at[pl.ds(base, b_per_w)])

    return k(table, idx)


def kernel(inData, inIndex, inShape):
    nbatch = inIndex.shape[0]
    t = inData.shape[0]
    t_pad = (t + 2 * _LANES - 1) // (2 * _LANES) * (2 * _LANES)
    table = jnp.pad(inData.astype(jnp.float32), (0, t_pad - t))
    idx = inIndex.astype(jnp.int32)
    out = _sc_gather(table, idx)
    return out.reshape((nbatch,) + (1,) * (len(inShape) - 1))


# speedup vs baseline: 4.6961x; 1.0277x over previous
import functools

import jax
import jax.numpy as jnp
from jax import lax
from jax.experimental import pallas as pl
from jax.experimental.pallas import tpu as pltpu
from jax.experimental.pallas import tpu_sc as plsc

_LANES = 16


def _sc_gather(table, idx):
    B = idx.shape[0]
    T = table.shape[0]
    info = plsc.get_sparse_core_info()
    nc, ns = info.num_cores, info.num_subcores
    nw = nc * ns
    b_per_w = B // nw

    mesh = plsc.VectorSubcoreMesh(core_axis_name="c", subcore_axis_name="s")

    @functools.partial(
        pl.kernel,
        mesh=mesh,
        out_type=jax.ShapeDtypeStruct((B,), jnp.float32),
        compiler_params=pltpu.CompilerParams(needs_layout_passes=False),
        scratch_types=[
            pltpu.VMEM((T,), jnp.float32),
            pltpu.VMEM((b_per_w,), jnp.int32),
            pltpu.VMEM((b_per_w,), jnp.float32),
            pltpu.SemaphoreType.DMA,
            pltpu.SemaphoreType.DMA,
        ],
    )
    def k(table_hbm, idx_hbm, out_hbm, table_v, idx_v, out_v, sem_t, sem_i):
        wid = lax.axis_index("s") * nc + lax.axis_index("c")
        base = wid * b_per_w
        cp_t = pltpu.async_copy(table_hbm, table_v, sem_t)
        cp_i = pltpu.async_copy(idx_hbm.at[pl.ds(base, b_per_w)], idx_v, sem_i)
        cp_i.wait()
        cp_t.wait()
        for i in range(b_per_w // _LANES):
            ids = idx_v[pl.ds(i * _LANES, _LANES)]
            out_v[pl.ds(i * _LANES, _LANES)] = plsc.load_gather(table_v, [ids])
        pltpu.sync_copy(out_v, out_hbm.at[pl.ds(base, b_per_w)])

    return k(table, idx)


def kernel(inData, inIndex, inShape):
    nbatch = inIndex.shape[0]
    out = _sc_gather(inData.astype(jnp.float32), inIndex.astype(jnp.int32))
    return out.reshape((nbatch,) + (1,) * (len(inShape) - 1))


# speedup vs baseline: 5.0170x; 1.0683x over previous
import functools

import jax
import jax.numpy as jnp
from jax import lax
from jax.experimental import pallas as pl
from jax.experimental.pallas import tpu as pltpu
from jax.experimental.pallas import tpu_sc as plsc

_LANES = 16


def _sc_gather(table, idx):
    B = idx.shape[0]
    T = table.shape[0]
    info = plsc.get_sparse_core_info()
    nc, ns = 1, info.num_subcores
    nw = nc * ns
    b_per_w = B // nw

    mesh = plsc.VectorSubcoreMesh(
        core_axis_name="c", subcore_axis_name="s", num_cores=1
    )

    @functools.partial(
        pl.kernel,
        mesh=mesh,
        out_type=jax.ShapeDtypeStruct((B,), jnp.float32),
        compiler_params=pltpu.CompilerParams(needs_layout_passes=False),
        scratch_types=[
            pltpu.VMEM((T,), jnp.float32),
            pltpu.VMEM((b_per_w,), jnp.int32),
            pltpu.VMEM((b_per_w,), jnp.float32),
            pltpu.SemaphoreType.DMA,
            pltpu.SemaphoreType.DMA,
        ],
    )
    def k(table_hbm, idx_hbm, out_hbm, table_v, idx_v, out_v, sem_t, sem_i):
        wid = lax.axis_index("s") * nc + lax.axis_index("c")
        base = wid * b_per_w
        cp_t = pltpu.async_copy(table_hbm, table_v, sem_t)
        cp_i = pltpu.async_copy(idx_hbm.at[pl.ds(base, b_per_w)], idx_v, sem_i)
        cp_i.wait()
        cp_t.wait()
        for i in range(b_per_w // _LANES):
            ids = idx_v[pl.ds(i * _LANES, _LANES)]
            out_v[pl.ds(i * _LANES, _LANES)] = plsc.load_gather(table_v, [ids])
        pltpu.sync_copy(out_v, out_hbm.at[pl.ds(base, b_per_w)])

    return k(table, idx)


def kernel(inData, inIndex, inShape):
    nbatch = inIndex.shape[0]
    out = _sc_gather(inData.astype(jnp.float32), inIndex.astype(jnp.int32))
    return out.reshape((nbatch,) + (1,) * (len(inShape) - 1))


# speedup vs baseline: 5.0336x; 1.0033x over previous
import functools

import jax
import jax.numpy as jnp
from jax import lax
from jax.experimental import pallas as pl
from jax.experimental.pallas import tpu as pltpu
from jax.experimental.pallas import tpu_sc as plsc

_LANES = 16


def _sc_gather(table, idx):
    B = idx.shape[0]
    T = table.shape[0]
    info = plsc.get_sparse_core_info()
    nc, ns = 1, info.num_subcores
    nw = nc * ns
    b_per_w = B // nw

    mesh = plsc.VectorSubcoreMesh(
        core_axis_name="c", subcore_axis_name="s", num_cores=1
    )

    @functools.partial(
        pl.kernel,
        mesh=mesh,
        out_type=jax.ShapeDtypeStruct((B,), jnp.float32),
        compiler_params=pltpu.CompilerParams(needs_layout_passes=False),
        scratch_types=[
            pltpu.VMEM((T,), jnp.float32),
            pltpu.VMEM((b_per_w,), jnp.int32),
            pltpu.VMEM((b_per_w,), jnp.float32),
            pltpu.SemaphoreType.DMA,
            pltpu.SemaphoreType.DMA,
            pltpu.SemaphoreType.DMA,
            pltpu.SemaphoreType.DMA,
        ],
    )
    def k(table_hbm, idx_hbm, out_hbm, table_v, idx_v, out_v,
          sem_t, sem_i0, sem_i1, sem_o):
        wid = lax.axis_index("s") * nc + lax.axis_index("c")
        base = wid * b_per_w
        half = b_per_w // 2
        cp_t = pltpu.async_copy(table_hbm, table_v, sem_t)
        cp_i0 = pltpu.async_copy(
            idx_hbm.at[pl.ds(base, half)], idx_v.at[pl.ds(0, half)], sem_i0)
        cp_i1 = pltpu.async_copy(
            idx_hbm.at[pl.ds(base + half, half)],
            idx_v.at[pl.ds(half, half)], sem_i1)
        cp_i0.wait()
        cp_t.wait()
        for i in range(half // _LANES):
            ids = idx_v[pl.ds(i * _LANES, _LANES)]
            out_v[pl.ds(i * _LANES, _LANES)] = plsc.load_gather(table_v, [ids])
        cp_o0 = pltpu.async_copy(
            out_v.at[pl.ds(0, half)], out_hbm.at[pl.ds(base, half)], sem_o)
        cp_i1.wait()
        for i in range(half // _LANES, b_per_w // _LANES):
            ids = idx_v[pl.ds(i * _LANES, _LANES)]
            out_v[pl.ds(i * _LANES, _LANES)] = plsc.load_gather(table_v, [ids])
        cp_o1 = pltpu.async_copy(
            out_v.at[pl.ds(half, half)],
            out_hbm.at[pl.ds(base + half, half)], sem_o)
        cp_o0.wait()
        cp_o1.wait()

    return k(table, idx)


def kernel(inData, inIndex, inShape):
    nbatch = inIndex.shape[0]
    out = _sc_gather(inData.astype(jnp.float32), inIndex.astype(jnp.int32))
    return out.reshape((nbatch,) + (1,) * (len(inShape) - 1))
